# Initial kernel scaffold; baseline (speedup 1.0000x reference)
import jax
import jax.numpy as jnp
from jax.experimental import pallas as pl


def kernel(x, adapter_ids, ranks, a_cache, b_cache, W):
    raise NotImplementedError("write your pallas kernel here")


# speedup vs baseline: 4.1230x; 4.1230x over previous
import functools

import jax
import jax.numpy as jnp
from jax.experimental import pallas as pl
from jax.experimental.pallas import tpu as pltpu

_NUM_ADAPTERS = 8
_MAX_RANK = 64
_TB = 512


def _lora_kernel(ids_ref, ranks_ref, x_ref, a_ref, b_ref, w_ref, out_ref):
    x = x_ref[...]
    base = jax.lax.dot_general(
        x, w_ref[...], (((1,), (1,)), ((), ())),
        preferred_element_type=jnp.float32)
    out_ref[...] = base

    ids = ids_ref[0]
    lo = jnp.min(ids)
    hi = jnp.max(ids)

    for e in range(_NUM_ADAPTERS):
        @pl.when(jnp.logical_and(lo <= e, e <= hi))
        def _():
            rank_mask = (jax.lax.broadcasted_iota(jnp.int32, (1, _MAX_RANK), 1)
                         < ranks_ref[e]).astype(jnp.float32)
            xa = jax.lax.dot_general(
                x, a_ref[e], (((1,), (1,)), ((), ())),
                preferred_element_type=jnp.float32)
            xa = xa * rank_mask
            contrib = jax.lax.dot_general(
                xa, b_ref[e], (((1,), (1,)), ((), ())),
                preferred_element_type=jnp.float32)
            row_mask = (ids == e).astype(jnp.float32)
            out_ref[...] += row_mask * contrib


@functools.partial(jax.jit, static_argnames=())
def kernel(x, adapter_ids, ranks, a_cache, b_cache, W):
    tok, d_in = x.shape
    d_out = W.shape[0]
    nb = tok // _TB
    ids = adapter_ids.astype(jnp.int32).reshape(nb, _TB, 1)
    ranks32 = ranks.astype(jnp.int32)

    grid_spec = pltpu.PrefetchScalarGridSpec(
        num_scalar_prefetch=0,
        grid=(nb,),
        in_specs=[
            pl.BlockSpec((1, _TB, 1), lambda i: (i, 0, 0)),
            pl.BlockSpec(memory_space=pltpu.SMEM),
            pl.BlockSpec((_TB, d_in), lambda i: (i, 0)),
            pl.BlockSpec((_NUM_ADAPTERS, _MAX_RANK, d_in), lambda i: (0, 0, 0)),
            pl.BlockSpec((_NUM_ADAPTERS, d_out, _MAX_RANK), lambda i: (0, 0, 0)),
            pl.BlockSpec((d_out, d_in), lambda i: (0, 0)),
        ],
        out_specs=pl.BlockSpec((_TB, d_out), lambda i: (i, 0)),
    )

    out = pl.pallas_call(
        _lora_kernel,
        grid_spec=grid_spec,
        out_shape=jax.ShapeDtypeStruct((tok, d_out), jnp.float32),
        compiler_params=pltpu.CompilerParams(
            dimension_semantics=("arbitrary",),
        ),
    )(ids, ranks32, x, a_cache, b_cache, W)
    return out
